# baseline (speedup 1.0000x reference)
import functools

import jax
import jax.numpy as jnp
from jax import lax
from jax.experimental import pallas as pl
from jax.experimental.pallas import tpu as pltpu
from jax.experimental.pallas import tpu_sc as plsc

B = 4096
Q_LEN = 20
D_LEN = 200
EDIM = 32

NC = 2
NS = 16
NW = NC * NS
BPW = B // NW

_mesh = plsc.VectorSubcoreMesh(core_axis_name="c", subcore_axis_name="s")


def _transpose_block(rows_ref, stage_ref, length, nchunks):
    ri = lax.iota(jnp.int32, 16)
    last = length - 16

    def chunk_body(j, carry):
        l0 = jnp.minimum(j * 16, last)
        rows_idx = l0 + ri
        for d in range(EDIM):
            col = jnp.full((16,), d, jnp.int32)
            v = plsc.load_gather(rows_ref, [rows_idx, col])
            stage_ref[d, pl.ds(l0, 16)] = v
        return carry

    lax.fori_loop(0, nchunks, chunk_body, 0)


@functools.partial(
    pl.kernel,
    mesh=_mesh,
    out_type=[
        jax.ShapeDtypeStruct((B, EDIM, Q_LEN), jnp.float32),
        jax.ShapeDtypeStruct((B, EDIM, D_LEN), jnp.float32),
    ],
    scratch_types=[
        pltpu.VMEM((BPW, Q_LEN), jnp.int32),
        pltpu.VMEM((BPW, 2, D_LEN // 2), jnp.int32),
        pltpu.VMEM((Q_LEN, EDIM), jnp.float32),
        pltpu.VMEM((D_LEN, EDIM), jnp.float32),
        pltpu.VMEM((EDIM, Q_LEN), jnp.float32),
        pltpu.VMEM((EDIM, D_LEN), jnp.float32),
        pltpu.SemaphoreType.DMA,
    ],
    compiler_params=pltpu.CompilerParams(
        needs_layout_passes=False, use_tc_tiling_on_sc=False),
)
def _emb_kernel(q_idx_hbm, d_idx_hbm, table_hbm, q_out_hbm, d_out_hbm,
                qidx_v, didx_v, qrows_v, drows_v, qstage_v, dstage_v, sem):
    wid = lax.axis_index("s") * NC + lax.axis_index("c")
    b0 = wid * BPW

    pltpu.sync_copy(q_idx_hbm.at[pl.ds(b0, BPW)], qidx_v)
    pltpu.sync_copy(d_idx_hbm.at[pl.ds(b0, BPW)], didx_v)

    half = D_LEN // 2

    def body(i, carry):
        bi = b0 + i
        cp_d0 = pltpu.async_copy(
            table_hbm.at[didx_v.at[i, 0]], drows_v.at[pl.ds(0, half)], sem)
        cp_d1 = pltpu.async_copy(
            table_hbm.at[didx_v.at[i, 1]], drows_v.at[pl.ds(half, half)], sem)
        cp_q = pltpu.async_copy(table_hbm.at[qidx_v.at[i]], qrows_v, sem)
        cp_d0.wait()
        cp_d1.wait()
        cp_q.wait()

        _transpose_block(drows_v, dstage_v, D_LEN, 13)
        _transpose_block(qrows_v, qstage_v, Q_LEN, 2)

        pltpu.sync_copy(dstage_v, d_out_hbm.at[bi])
        pltpu.sync_copy(qstage_v, q_out_hbm.at[bi])
        return carry

    lax.fori_loop(0, BPW, body, 0)


def kernel(query_input, document_input, table):
    d_idx = document_input.reshape(B, 2, D_LEN // 2)
    q_out, d_out = _emb_kernel(query_input, d_idx, table)
    return (q_out, d_out)


# speedup vs baseline: 1.1059x; 1.1059x over previous
import functools

import jax
import jax.numpy as jnp
from jax import lax
from jax.experimental import pallas as pl
from jax.experimental.pallas import tpu as pltpu
from jax.experimental.pallas import tpu_sc as plsc

B = 4096
Q_LEN = 20
D_LEN = 200
EDIM = 32
HALF = D_LEN // 2

NC = 2
NS = 16
NW = NC * NS
BPW = B // NW

NBUF = 4
NSTEP = BPW // NBUF

_mesh = plsc.VectorSubcoreMesh(core_axis_name="c", subcore_axis_name="s")


def _transpose_block(rows_ref, stage_ref, length, nchunks):
    ri = lax.iota(jnp.int32, 16)
    last = length - 16

    def chunk_body(j, carry):
        l0 = jnp.minimum(j * 16, last)
        rows_idx = l0 + ri
        for d in range(EDIM):
            col = jnp.full((16,), d, jnp.int32)
            v = plsc.load_gather(rows_ref, [rows_idx, col])
            stage_ref[d, pl.ds(l0, 16)] = v
        return carry

    lax.fori_loop(0, nchunks, chunk_body, 0)


@functools.partial(
    pl.kernel,
    mesh=_mesh,
    out_type=[
        jax.ShapeDtypeStruct((B, EDIM, Q_LEN), jnp.float32),
        jax.ShapeDtypeStruct((B, EDIM, D_LEN), jnp.float32),
    ],
    scratch_types=[
        pltpu.VMEM((BPW, Q_LEN), jnp.int32),
        pltpu.VMEM((BPW, 2, HALF), jnp.int32),
        pltpu.VMEM((NBUF, Q_LEN, EDIM), jnp.float32),
        pltpu.VMEM((NBUF, D_LEN, EDIM), jnp.float32),
        pltpu.VMEM((NBUF, EDIM, Q_LEN), jnp.float32),
        pltpu.VMEM((NBUF, EDIM, D_LEN), jnp.float32),
        [pltpu.SemaphoreType.DMA] * NBUF,
        [pltpu.SemaphoreType.DMA] * NBUF,
    ],
    compiler_params=pltpu.CompilerParams(
        needs_layout_passes=False, use_tc_tiling_on_sc=False),
)
def _emb_kernel(q_idx_hbm, d_idx_hbm, table_hbm, q_out_hbm, d_out_hbm,
                qidx_v, didx_v, qrows_v, drows_v, qstage_v, dstage_v,
                gsems, osems):
    wid = lax.axis_index("s") * NC + lax.axis_index("c")
    b0 = wid * BPW

    pltpu.sync_copy(q_idx_hbm.at[pl.ds(b0, BPW)], qidx_v)
    pltpu.sync_copy(d_idx_hbm.at[pl.ds(b0, BPW)], didx_v)

    def issue_gathers(i, s):
        pltpu.async_copy(
            table_hbm.at[didx_v.at[i, 0]],
            drows_v.at[s, pl.ds(0, HALF)], gsems[s])
        pltpu.async_copy(
            table_hbm.at[didx_v.at[i, 1]],
            drows_v.at[s, pl.ds(HALF, HALF)], gsems[s])
        pltpu.async_copy(table_hbm.at[qidx_v.at[i]], qrows_v.at[s], gsems[s])

    def wait_gathers(s):
        pltpu.make_async_copy(
            table_hbm.at[didx_v.at[0, 0]],
            drows_v.at[s, pl.ds(0, HALF)], gsems[s]).wait()
        pltpu.make_async_copy(
            table_hbm.at[didx_v.at[0, 1]],
            drows_v.at[s, pl.ds(HALF, HALF)], gsems[s]).wait()
        pltpu.make_async_copy(
            table_hbm.at[qidx_v.at[0]], qrows_v.at[s], gsems[s]).wait()

    def wait_outs(s):
        pltpu.make_async_copy(
            dstage_v.at[s], d_out_hbm.at[b0], osems[s]).wait()
        pltpu.make_async_copy(
            qstage_v.at[s], q_out_hbm.at[b0], osems[s]).wait()

    for s in range(NBUF):
        issue_gathers(s, s)

    def body(j, carry):
        for s in range(NBUF):
            i = j * NBUF + s
            bi = b0 + i
            wait_gathers(s)
            @pl.when(j > 0)
            def _():
                wait_outs(s)
            _transpose_block(drows_v.at[s], dstage_v.at[s], D_LEN, 13)
            _transpose_block(qrows_v.at[s], qstage_v.at[s], Q_LEN, 2)
            issue_gathers(jnp.minimum(i + NBUF, BPW - 1), s)
            pltpu.async_copy(dstage_v.at[s], d_out_hbm.at[bi], osems[s])
            pltpu.async_copy(qstage_v.at[s], q_out_hbm.at[bi], osems[s])
        return carry

    lax.fori_loop(0, NSTEP, body, 0)

    for s in range(NBUF):
        wait_gathers(s)
        wait_outs(s)


def kernel(query_input, document_input, table):
    d_idx = document_input.reshape(B, 2, HALF)
    q_out, d_out = _emb_kernel(query_input, d_idx, table)
    return (q_out, d_out)


# speedup vs baseline: 1.4764x; 1.3350x over previous
import functools

import jax
import jax.numpy as jnp
from jax import lax
from jax.experimental import pallas as pl
from jax.experimental.pallas import tpu as pltpu
from jax.experimental.pallas import tpu_sc as plsc

B = 4096
Q_LEN = 20
D_LEN = 200
EDIM = 32

NC = 2
NS = 16
NW = NC * NS
GB = B // NW
QT = EDIM // 8
DT = D_LEN // 8
NR = 4

_mesh = plsc.VectorSubcoreMesh(core_axis_name="c", subcore_axis_name="s")


@functools.partial(
    pl.kernel,
    mesh=_mesh,
    out_type=[
        jax.ShapeDtypeStruct((Q_LEN, QT, NW, 8, GB), jnp.float32),
        jax.ShapeDtypeStruct((EDIM, DT, NW, 8, GB), jnp.float32),
    ],
    scratch_types=[
        pltpu.VMEM((Q_LEN, GB), jnp.int32),
        pltpu.VMEM((D_LEN, GB), jnp.int32),
        pltpu.VMEM((NR, GB, EDIM), jnp.float32),
        pltpu.VMEM((2, QT, 8, GB), jnp.float32),
        pltpu.VMEM((2, EDIM, 8, GB), jnp.float32),
        [pltpu.SemaphoreType.DMA] * NR,
        [pltpu.SemaphoreType.DMA] * 2,
        [pltpu.SemaphoreType.DMA] * 2,
    ],
    compiler_params=pltpu.CompilerParams(
        needs_layout_passes=False, use_tc_tiling_on_sc=False),
)
def _emb_kernel(qT_hbm, dT_hbm, table_hbm, q_out_hbm, d_out_hbm,
                qidx_v, didx_v, rows_v, qstage_v, dstage_v,
                gsems, qosems, dosems):
    wid = lax.axis_index("s") * NC + lax.axis_index("c")
    b0 = wid * GB
    ri = lax.iota(jnp.int32, 16)

    pltpu.sync_copy(qT_hbm.at[:, pl.ds(b0, GB)], qidx_v)
    pltpu.sync_copy(dT_hbm.at[:, pl.ds(b0, GB)], didx_v)

    def issue_q(l, r):
        pltpu.async_copy(table_hbm.at[qidx_v.at[l]], rows_v.at[r], gsems[r])

    def issue_d(l, r):
        pltpu.async_copy(table_hbm.at[didx_v.at[l]], rows_v.at[r], gsems[r])

    def wait_g(r):
        pltpu.make_async_copy(
            table_hbm.at[qidx_v.at[0]], rows_v.at[r], gsems[r]).wait()

    def transpose_into(r, store):
        def cbody(c, carry):
            bi = c * 16 + ri
            for d in range(EDIM):
                col = jnp.full((16,), d, jnp.int32)
                v = plsc.load_gather(rows_v.at[r], [bi, col])
                store(d, c * 16, v)
            return carry
        lax.fori_loop(0, GB // 16, cbody, 0)

    issue_q(0, 0)
    issue_q(1, 1)

    def qbody(j, carry):
        for s in range(2):
            l = 2 * j + s
            wait_g(s)

            @pl.when(j > 0)
            def _():
                pltpu.make_async_copy(
                    qstage_v.at[s], q_out_hbm.at[0, :, 0], qosems[s]).wait()

            qstage = qstage_v.at[s]
            transpose_into(
                s, lambda d, c0, v: qstage.__setitem__(
                    (d // 8, d % 8, pl.ds(c0, 16)), v))
            issue_q(jnp.minimum(l + 2, Q_LEN - 1), s)
            pltpu.async_copy(
                qstage_v.at[s], q_out_hbm.at[l, :, wid], qosems[s])
        return carry

    lax.fori_loop(0, Q_LEN // 2, qbody, 0)
    for s in range(2):
        wait_g(s)
        pltpu.make_async_copy(
            qstage_v.at[s], q_out_hbm.at[0, :, 0], qosems[s]).wait()

    for r in range(NR):
        issue_d(r, r)

    def wait_do(sd):
        pltpu.make_async_copy(
            dstage_v.at[sd], d_out_hbm.at[:, 0, 0], dosems[sd]).wait()

    def do_tile(lt, sd):
        dstage = dstage_v.at[sd]
        for li in range(8):
            r = li % NR
            l = lt * 8 + li
            wait_g(r)
            transpose_into(
                r, lambda d, c0, v: dstage.__setitem__(
                    (d, li, pl.ds(c0, 16)), v))
            issue_d(jnp.minimum(l + NR, D_LEN - 1), r)
        pltpu.async_copy(
            dstage_v.at[sd], d_out_hbm.at[:, lt, wid], dosems[sd])

    def dbody(j, carry):
        for sd in range(2):
            @pl.when(j > 0)
            def _():
                wait_do(sd)
            do_tile(2 * j + sd, sd)
        return carry

    lax.fori_loop(0, DT // 2, dbody, 0)
    wait_do(0)
    do_tile(DT - 1, 0)

    wait_do(0)
    wait_do(1)
    for r in range(NR):
        wait_g(r)


def kernel(query_input, document_input, table):
    qT = jnp.transpose(query_input)
    dT = jnp.transpose(document_input)
    QO, DO = _emb_kernel(qT, dT, table)
    q_out = jnp.transpose(QO, (2, 4, 1, 3, 0)).reshape(B, EDIM, Q_LEN)
    d_out = jnp.transpose(DO, (2, 4, 0, 1, 3)).reshape(B, EDIM, D_LEN)
    return (q_out, d_out)
